# baseline (speedup 1.0000x reference)
import functools

import jax
import jax.numpy as jnp
from jax import lax
from jax.experimental import pallas as pl
from jax.experimental.pallas import tpu as pltpu
from jax.experimental.pallas import tpu_sc as plsc

_INFO = plsc.get_sparse_core_info()
_NC, _NS = _INFO.num_cores, _INFO.num_subcores
_NW = _NC * _NS

BATCH = 16384
FEAT = 32
_B_PER_W = BATCH // _NW
_DEPTH = 8
_NGROUPS = _B_PER_W // 16


@functools.partial(
    pl.kernel,
    mesh=plsc.VectorSubcoreMesh(core_axis_name="c", subcore_axis_name="s"),
    out_type=jax.ShapeDtypeStruct((FEAT, BATCH), jnp.float32),
    scratch_types=[
        pltpu.VMEM((_B_PER_W + 16,), jnp.int32),
        pltpu.VMEM((_DEPTH, FEAT, 128), jnp.float32),
        pltpu.VMEM((FEAT, _B_PER_W), jnp.float32),
        pltpu.SemaphoreType.DMA((_DEPTH,)),
    ],
    compiler_params=pltpu.CompilerParams(needs_layout_passes=False),
)
def _gather_kernel(tablet_hbm, idx_hbm, out_hbm, idx_v, blk_v, cols_v, sem):
    wid = lax.axis_index("s") * _NC + lax.axis_index("c")
    base = wid * _B_PER_W
    pltpu.sync_copy(idx_hbm.at[pl.ds(base, _B_PER_W)], idx_v.at[pl.ds(0, _B_PER_W)])
    idx_v[pl.ds(_B_PER_W, 16)] = jnp.zeros((16,), jnp.int32)

    iota16 = lax.iota(jnp.int32, 16)

    def fire(i, bank):
        start = pl.multiple_of((i // 128) * 128, 128)
        pltpu.make_async_copy(
            tablet_hbm.at[:, pl.ds(start, 128)], blk_v.at[bank], sem.at[bank]
        ).start()

    def drain(bank):
        pltpu.make_async_copy(
            tablet_hbm.at[:, pl.ds(0, 128)], blk_v.at[bank], sem.at[bank]
        ).wait()

    vec0 = idx_v[pl.ds(0, 16)]
    for k in range(_DEPTH):
        fire(vec0[k], k)

    def body(g, carry):
        vec = idx_v[pl.ds(g * 16, 16)]
        vecn = idx_v[pl.ds(g * 16 + 16, 16)]
        for k in range(16):
            e = g * 16 + k
            bank = k % _DEPTH
            drain(bank)
            c_vec = jnp.full((16,), lax.rem(vec[k], 128), jnp.int32)
            b_vec = jnp.full((16,), bank, jnp.int32)
            e_vec = jnp.full((16,), e, jnp.int32)
            va = plsc.load_gather(blk_v, [b_vec, iota16, c_vec])
            vb = plsc.load_gather(blk_v, [b_vec, iota16 + 16, c_vec])
            plsc.store_scatter(cols_v, [iota16, e_vec], va)
            plsc.store_scatter(cols_v, [iota16 + 16, e_vec], vb)
            nk = k + _DEPTH
            fire(vec[nk] if nk < 16 else vecn[nk - 16], bank)
        return carry

    lax.fori_loop(0, _NGROUPS, body, 0)
    for k in range(_DEPTH):
        drain(k)
    pltpu.sync_copy(cols_v, out_hbm.at[:, pl.ds(base, _B_PER_W)])


def kernel(idx, representations):
    out_t = _gather_kernel(representations.T, idx.astype(jnp.int32))
    return out_t.T
